# baseline (speedup 1.0000x reference)
import jax
import jax.numpy as jnp
from jax.experimental import pallas as pl
from jax.experimental.pallas import tpu as pltpu

N_CATEGORY = 20
N_TYPE = 200
N_VARIANT = 800
N_SPATIAL = 20
D = 320
TOTAL = N_CATEGORY + N_TYPE + N_VARIANT + N_SPATIAL

W_CAT = N_CATEGORY * D
W_TYPE = N_TYPE * D
W_VAR = N_VARIANT * D
W_SPA = N_SPATIAL * D


def _concat_dma_body(cat_ref, typ_ref, var_ref, spa_ref, out_ref,
                     sem_cat, sem_typ, sem_var, sem_spa):
    copies = [
        pltpu.make_async_copy(cat_ref, out_ref.at[pl.ds(0, W_CAT)], sem_cat),
        pltpu.make_async_copy(typ_ref, out_ref.at[pl.ds(W_CAT, W_TYPE)], sem_typ),
        pltpu.make_async_copy(var_ref, out_ref.at[pl.ds(W_CAT + W_TYPE, W_VAR)], sem_var),
        pltpu.make_async_copy(spa_ref, out_ref.at[pl.ds(W_CAT + W_TYPE + W_VAR, W_SPA)], sem_spa),
    ]
    for c in copies:
        c.start()
    for c in copies:
        c.wait()


def kernel(category_codes, type_codes, variant_codes, spatial_codes):
    flat = pl.pallas_call(
        _concat_dma_body,
        out_shape=jax.ShapeDtypeStruct((TOTAL * D,), jnp.float32),
        in_specs=[pl.BlockSpec(memory_space=pl.ANY)] * 4,
        out_specs=pl.BlockSpec(memory_space=pl.ANY),
        scratch_shapes=[pltpu.SemaphoreType.DMA] * 4,
    )(
        category_codes.reshape(W_CAT),
        type_codes.reshape(W_TYPE),
        variant_codes.reshape(W_VAR),
        spatial_codes.reshape(W_SPA),
    )
    return flat.reshape(TOTAL, D)


# speedup vs baseline: 1.7971x; 1.7971x over previous
import functools
import jax
import jax.numpy as jnp
from jax import lax
from jax.experimental import pallas as pl
from jax.experimental.pallas import tpu as pltpu
from jax.experimental.pallas import tpu_sc as plsc

D = 320
N_CAT, N_TYPE, N_VAR, N_SPA = 20, 200, 800, 20
TOTAL = N_CAT + N_TYPE + N_VAR + N_SPA
W_CAT, W_TYPE, W_VAR, W_SPA = N_CAT * D, N_TYPE * D, N_VAR * D, N_SPA * D
NW = 32
WORDS_PER_W = (W_CAT + W_TYPE + W_VAR + W_SPA) // NW

_mesh = plsc.VectorSubcoreMesh(core_axis_name="c", subcore_axis_name="s")


@functools.partial(
    pl.kernel,
    mesh=_mesh,
    out_type=jax.ShapeDtypeStruct((TOTAL * D,), jnp.float32),
    scratch_types=[
        pltpu.VMEM((WORDS_PER_W,), jnp.float32),
        pltpu.SemaphoreType.DMA,
        pltpu.SemaphoreType.DMA,
    ],
)
def _flatten_sc(cat, typ, var, spa, out, buf, sem_in, sem_out):
    wid = lax.axis_index("s") * 2 + lax.axis_index("c")
    plan = []
    voff = 0
    for src, base, n in (
        (cat, 0, W_CAT),
        (typ, W_CAT, W_TYPE),
        (var, W_CAT + W_TYPE, W_VAR),
        (spa, W_CAT + W_TYPE + W_VAR, W_SPA),
    ):
        cw = n // NW
        plan.append((src, base, cw, voff))
        voff += cw
    ins = [
        pltpu.make_async_copy(
            src.at[pl.ds(wid * cw, cw)], buf.at[pl.ds(vo, cw)], sem_in
        )
        for src, base, cw, vo in plan
    ]
    outs = [
        pltpu.make_async_copy(
            buf.at[pl.ds(vo, cw)], out.at[pl.ds(base + wid * cw, cw)], sem_out
        )
        for src, base, cw, vo in plan
    ]
    for c in ins:
        c.start()
    for c in ins:
        c.wait()
    for c in outs:
        c.start()
    for c in outs:
        c.wait()


def kernel(category_codes, type_codes, variant_codes, spatial_codes):
    flat = _flatten_sc(
        category_codes.reshape(-1),
        type_codes.reshape(-1),
        variant_codes.reshape(-1),
        spatial_codes.reshape(-1),
    )
    return flat.reshape(TOTAL, D)


# speedup vs baseline: 5.6578x; 3.1482x over previous
import jax
import jax.numpy as jnp
from jax.experimental import pallas as pl

N_CATEGORY = 20
N_TYPE = 200
N_VARIANT = 800
N_SPATIAL = 20
D = 320
TOTAL = N_CATEGORY + N_TYPE + N_VARIANT + N_SPATIAL


def _concat_body(cat_ref, typ_ref, var_ref, spa_ref, out_ref):
    out_ref[...] = jnp.concatenate(
        [cat_ref[...], typ_ref[...], var_ref[...], spa_ref[...]], axis=0
    )


def kernel(category_codes, type_codes, variant_codes, spatial_codes):
    typ = type_codes.reshape(N_TYPE, D)
    var = variant_codes.reshape(N_VARIANT, D)
    return pl.pallas_call(
        _concat_body,
        out_shape=jax.ShapeDtypeStruct((TOTAL, D), jnp.float32),
    )(category_codes, typ, var, spatial_codes)
